# Initial kernel scaffold; baseline (speedup 1.0000x reference)
import jax
import jax.numpy as jnp
from jax.experimental import pallas as pl


def kernel(bert_outputs, token_idxs, token_masks, token_nums, W1, b1, W2, b2, transitions, start_trans, end_trans):
    raise NotImplementedError("write your pallas kernel here")


# speedup vs baseline: 4.1786x; 4.1786x over previous
import functools

import jax
import jax.numpy as jnp
from jax import lax
from jax.experimental import pallas as pl
from jax.experimental.pallas import tpu as pltpu
from jax.experimental.pallas import tpu_sc as plsc

_NC, _NS = 2, 16
_NW = _NC * _NS
_CH = 32
_KP = 128
_NEG = -1e30


def _sc_gather(table, idx2, TW, L):
    R, D = table.shape
    rpw = R // _NW
    chunks = rpw // _CH
    mesh = plsc.VectorSubcoreMesh(core_axis_name="c", subcore_axis_name="s")

    @functools.partial(
        pl.kernel,
        mesh=mesh,
        out_type=jax.ShapeDtypeStruct((R, D), jnp.float32),
        scratch_types=[
            pltpu.VMEM((rpw // _CH, _CH), jnp.int32),
            pltpu.VMEM((_CH, D), jnp.float32),
            pltpu.SemaphoreType.DMA,
        ],
    )
    def k(table_hbm, idx_hbm, out_hbm, idx_v, rows_v, sem):
        wid = lax.axis_index("s") * _NC + lax.axis_index("c")
        base = wid * rpw
        pltpu.sync_copy(idx_hbm.at[pl.ds(wid * chunks, chunks)], idx_v)
        off = (base // TW) * L
        for r in range(chunks):
            for i in range(_CH // 16):
                sl = pl.ds(i * 16, 16)
                idx_v[r, sl] = idx_v[r, sl] + off
        for c in range(chunks):
            pltpu.async_copy(table_hbm.at[idx_v.at[c]], rows_v, sem).wait()
            pltpu.sync_copy(rows_v, out_hbm.at[pl.ds(base + c * _CH, _CH)])

    return k(table, idx2)


def _mlp_body(g_ref, m0_ref, m1_ref, w1_ref, b1_ref, w2_ref, b2_ref, o_ref):
    D = w1_ref.shape[0]
    g = g_ref[...]
    tr = g[:, :D] * m0_ref[...] + g[:, D:] * m1_ref[...]
    h = jnp.dot(tr, w1_ref[...], preferred_element_type=jnp.float32)
    h = jnp.maximum(h + b1_ref[...], 0.0)
    o_ref[...] = jnp.dot(h, w2_ref[...], preferred_element_type=jnp.float32) + b2_ref[...]


def _mlp(g2, m0, m1, W1, b1r, W2p, b2r, blk=256):
    NR, D2 = g2.shape
    D, H = W1.shape
    grid = NR // blk
    return pl.pallas_call(
        _mlp_body,
        grid=(grid,),
        in_specs=[
            pl.BlockSpec((blk, D2), lambda i: (i, 0)),
            pl.BlockSpec((blk, 1), lambda i: (i, 0)),
            pl.BlockSpec((blk, 1), lambda i: (i, 0)),
            pl.BlockSpec((D, H), lambda i: (0, 0)),
            pl.BlockSpec((1, H), lambda i: (0, 0)),
            pl.BlockSpec((H, _KP), lambda i: (0, 0)),
            pl.BlockSpec((1, _KP), lambda i: (0, 0)),
        ],
        out_specs=pl.BlockSpec((blk, _KP), lambda i: (i, 0)),
        out_shape=jax.ShapeDtypeStruct((NR, _KP), jnp.float32),
    )(g2, m0, m1, W1, b1r, W2p, b2r)


def _crf_body(K, s_ref, tr_ref, st_ref, en_ref, ln_ref, o_ref):
    T = s_ref.shape[0]
    ii = lax.broadcasted_iota(jnp.int32, (_KP, _KP), 0)
    jj = lax.broadcasted_iota(jnp.int32, (_KP, _KP), 1)
    E = jnp.where((ii < K) & (jj < K), jnp.exp(tr_ref[...]), 0.0)
    a0 = s_ref[0] + st_ref[...]
    c = jnp.max(a0, axis=1, keepdims=True)
    u = jnp.exp(a0 - c)
    ln = ln_ref[...]

    def step(t, carry):
        u, c = carry
        w = jnp.dot(u, E, preferred_element_type=jnp.float32) * jnp.exp(s_ref[t])
        s = jnp.max(w, axis=1, keepdims=True)
        un = w / s
        cn = c + jnp.log(s)
        keep = t.astype(jnp.float32) < ln
        return jnp.where(keep, un, u), jnp.where(keep, cn, c)

    u, c = lax.fori_loop(1, T, step, (u, c))
    z = jnp.sum(u * jnp.exp(en_ref[...]), axis=1, keepdims=True)
    o_ref[...] = c + jnp.log(z)


def _crf(s_t, trp, stp, enp, lnf, K):
    T, B, _ = s_t.shape
    return pl.pallas_call(
        functools.partial(_crf_body, K),
        out_shape=jax.ShapeDtypeStruct((B, 1), jnp.float32),
    )(s_t, trp, stp, enp, lnf)


def kernel(bert_outputs, token_idxs, token_masks, token_nums, W1, b1, W2, b2,
           transitions, start_trans, end_trans):
    B, L, D = bert_outputs.shape
    TW = token_idxs.shape[1]
    T = TW // 2
    K = W2.shape[1]
    kpad = _KP - K

    table = bert_outputs.reshape(B * L, D)
    idx2 = token_idxs.reshape(B * TW // _CH, _CH).astype(jnp.int32)
    gathered = _sc_gather(table, idx2, TW, L)

    g2 = gathered.reshape(B * T, 2 * D)
    m = token_masks.reshape(B * T, 2)
    W2p = jnp.pad(W2, ((0, 0), (0, kpad)))
    b2r = jnp.pad(b2, (0, kpad)).reshape(1, _KP)
    scores = _mlp(g2, m[:, 0:1], m[:, 1:2], W1, b1.reshape(1, -1), W2p, b2r)

    s_t = scores.reshape(B, T, _KP).transpose(1, 0, 2)
    trp = jnp.pad(transitions, ((0, kpad), (0, kpad)))
    stp = jnp.pad(start_trans, (0, kpad), constant_values=_NEG).reshape(1, _KP)
    enp = jnp.pad(end_trans, (0, kpad), constant_values=_NEG).reshape(1, _KP)
    lnf = jnp.maximum(token_nums, 1).astype(jnp.float32).reshape(B, 1)
    logZ = _crf(s_t, trp, stp, enp, lnf, K)
    return logZ[:, 0]


# speedup vs baseline: 4.2121x; 1.0080x over previous
import functools

import jax
import jax.numpy as jnp
from jax import lax
from jax.experimental import pallas as pl
from jax.experimental.pallas import tpu as pltpu
from jax.experimental.pallas import tpu_sc as plsc

_NC, _NS = 2, 16
_NW = _NC * _NS
_CH = 32
_KP = 128
_NEG = -1e30


def _sc_gather(table, idx2, TW, L):
    R, D = table.shape
    rpw = R // _NW
    chunks = rpw // _CH
    mesh = plsc.VectorSubcoreMesh(core_axis_name="c", subcore_axis_name="s")

    @functools.partial(
        pl.kernel,
        mesh=mesh,
        out_type=jax.ShapeDtypeStruct((R, D), jnp.float32),
        scratch_types=[
            pltpu.VMEM((rpw // _CH, _CH), jnp.int32),
            pltpu.VMEM((_CH, D), jnp.float32),
            pltpu.SemaphoreType.DMA,
        ],
    )
    def k(table_hbm, idx_hbm, out_hbm, idx_v, rows_v, sem):
        wid = lax.axis_index("s") * _NC + lax.axis_index("c")
        base = wid * rpw
        pltpu.sync_copy(idx_hbm.at[pl.ds(wid * chunks, chunks)], idx_v)
        off = (base // TW) * L
        for r in range(chunks):
            for i in range(_CH // 16):
                sl = pl.ds(i * 16, 16)
                idx_v[r, sl] = idx_v[r, sl] + off
        for c in range(chunks):
            pltpu.async_copy(table_hbm.at[idx_v.at[c]], rows_v, sem).wait()
            pltpu.sync_copy(rows_v, out_hbm.at[pl.ds(base + c * _CH, _CH)])

    return k(table, idx2)


def _mlp_body(g_ref, m0_ref, m1_ref, w1_ref, b1_ref, w2_ref, b2_ref, o_ref):
    D = w1_ref.shape[0]
    g = g_ref[...]
    tr = g[:, :D] * m0_ref[...] + g[:, D:] * m1_ref[...]
    h = jnp.dot(tr.astype(jnp.bfloat16), w1_ref[...],
                preferred_element_type=jnp.float32)
    h = jnp.maximum(h + b1_ref[...], 0.0)
    o_ref[...] = jnp.dot(h.astype(jnp.bfloat16), w2_ref[...],
                         preferred_element_type=jnp.float32) + b2_ref[...]


def _mlp(g2, m0, m1, W1, b1r, W2p, b2r, blk=256):
    NR, D2 = g2.shape
    D, H = W1.shape
    grid = NR // blk
    return pl.pallas_call(
        _mlp_body,
        grid=(grid,),
        in_specs=[
            pl.BlockSpec((blk, D2), lambda i: (i, 0)),
            pl.BlockSpec((blk, 1), lambda i: (i, 0)),
            pl.BlockSpec((blk, 1), lambda i: (i, 0)),
            pl.BlockSpec((D, H), lambda i: (0, 0)),
            pl.BlockSpec((1, H), lambda i: (0, 0)),
            pl.BlockSpec((H, _KP), lambda i: (0, 0)),
            pl.BlockSpec((1, _KP), lambda i: (0, 0)),
        ],
        out_specs=pl.BlockSpec((blk, _KP), lambda i: (i, 0)),
        out_shape=jax.ShapeDtypeStruct((NR, _KP), jnp.float32),
    )(g2, m0, m1, W1, b1r, W2p, b2r)


def _crf_body(K, s_ref, tr_ref, st_ref, en_ref, ln_ref, o_ref):
    T = s_ref.shape[0]
    ii = lax.broadcasted_iota(jnp.int32, (_KP, _KP), 0)
    jj = lax.broadcasted_iota(jnp.int32, (_KP, _KP), 1)
    E = jnp.where((ii < K) & (jj < K), jnp.exp(tr_ref[...]), 0.0)
    a0 = s_ref[0] + st_ref[...]
    c = jnp.max(a0, axis=1, keepdims=True)
    u = jnp.exp(a0 - c)
    ln = ln_ref[...]

    def step(t, carry):
        u, c = carry
        w = jnp.dot(u, E, preferred_element_type=jnp.float32) * jnp.exp(s_ref[t])
        s = jnp.max(w, axis=1, keepdims=True)
        un = w / s
        cn = c + jnp.log(s)
        keep = t.astype(jnp.float32) < ln
        return jnp.where(keep, un, u), jnp.where(keep, cn, c)

    u, c = lax.fori_loop(1, T, step, (u, c))
    z = jnp.sum(u * jnp.exp(en_ref[...]), axis=1, keepdims=True)
    o_ref[...] = c + jnp.log(z)


def _crf(s_t, trp, stp, enp, lnf, K):
    T, B, _ = s_t.shape
    return pl.pallas_call(
        functools.partial(_crf_body, K),
        out_shape=jax.ShapeDtypeStruct((B, 1), jnp.float32),
    )(s_t, trp, stp, enp, lnf)


def kernel(bert_outputs, token_idxs, token_masks, token_nums, W1, b1, W2, b2,
           transitions, start_trans, end_trans):
    B, L, D = bert_outputs.shape
    TW = token_idxs.shape[1]
    T = TW // 2
    K = W2.shape[1]
    kpad = _KP - K

    table = bert_outputs.reshape(B * L, D)
    idx2 = token_idxs.reshape(B * TW // _CH, _CH).astype(jnp.int32)
    gathered = _sc_gather(table, idx2, TW, L)

    g2 = gathered.reshape(B * T, 2 * D)
    m = token_masks.reshape(B * T, 2)
    W2p = jnp.pad(W2, ((0, 0), (0, kpad))).astype(jnp.bfloat16)
    b2r = jnp.pad(b2, (0, kpad)).reshape(1, _KP)
    scores = _mlp(g2, m[:, 0:1], m[:, 1:2], W1.astype(jnp.bfloat16),
                  b1.reshape(1, -1), W2p, b2r)

    s_t = scores.reshape(B, T, _KP).transpose(1, 0, 2)
    trp = jnp.pad(transitions, ((0, kpad), (0, kpad)))
    stp = jnp.pad(start_trans, (0, kpad), constant_values=_NEG).reshape(1, _KP)
    enp = jnp.pad(end_trans, (0, kpad), constant_values=_NEG).reshape(1, _KP)
    lnf = jnp.maximum(token_nums, 1).astype(jnp.float32).reshape(B, 1)
    logZ = _crf(s_t, trp, stp, enp, lnf, K)
    return logZ[:, 0]


# speedup vs baseline: 4.8824x; 1.1591x over previous
import functools

import jax
import jax.numpy as jnp
from jax import lax
from jax.experimental import pallas as pl
from jax.experimental.pallas import tpu as pltpu
from jax.experimental.pallas import tpu_sc as plsc

_NC, _NS = 2, 16
_NW = _NC * _NS
_CH = 32
_KP = 128
_NEG = -1e30
_GRP = 4


def _sc_gather(table, idx2, TW, L):
    R, D = table.shape
    rpw = R // _NW
    chunks = rpw // _CH
    mesh = plsc.VectorSubcoreMesh(core_axis_name="c", subcore_axis_name="s")

    @functools.partial(
        pl.kernel,
        mesh=mesh,
        out_type=jax.ShapeDtypeStruct((R, D), jnp.float32),
        scratch_types=[
            pltpu.VMEM((chunks, _CH), jnp.int32),
            pltpu.VMEM((2, _CH, D), jnp.float32),
            pltpu.SemaphoreType.DMA,
            pltpu.SemaphoreType.DMA,
            pltpu.SemaphoreType.DMA,
            pltpu.SemaphoreType.DMA,
        ],
    )
    def k(table_hbm, idx_hbm, out_hbm, idx_v, rows_v, g0, g1, o0, o1):
        gsem = (g0, g1)
        osem = (o0, o1)
        wid = lax.axis_index("s") * _NC + lax.axis_index("c")
        base = wid * rpw
        pltpu.sync_copy(idx_hbm.at[pl.ds(wid * chunks, chunks)], idx_v)
        off = (base // TW) * L
        for r in range(chunks):
            for i in range(_CH // 16):
                sl = pl.ds(i * 16, 16)
                idx_v[r, sl] = idx_v[r, sl] + off
        gather_h = [None] * chunks
        out_h = [None] * chunks

        def issue_gather(c):
            buf = c & 1
            gather_h[c] = pltpu.async_copy(
                table_hbm.at[idx_v.at[c]], rows_v.at[buf], gsem[buf])

        issue_gather(0)
        for c in range(chunks):
            buf = c & 1
            gather_h[c].wait()
            out_h[c] = pltpu.async_copy(
                rows_v.at[buf], out_hbm.at[pl.ds(base + c * _CH, _CH)],
                osem[buf])
            if c + 1 < chunks:
                if c >= 1:
                    out_h[c - 1].wait()
                issue_gather(c + 1)
        out_h[chunks - 1].wait()
        if chunks >= 2:
            out_h[chunks - 2].wait()

    return k(table, idx2)


def _mlp_body(g_ref, m0_ref, m1_ref, w1_ref, b1_ref, w2_ref, b2_ref, o_ref):
    D = w1_ref.shape[0]
    g = g_ref[...]
    tr = g[:, :D] * m0_ref[...] + g[:, D:] * m1_ref[...]
    h = jnp.dot(tr.astype(jnp.bfloat16), w1_ref[...],
                preferred_element_type=jnp.float32)
    h = jnp.maximum(h + b1_ref[...], 0.0)
    s = jnp.dot(h.astype(jnp.bfloat16), w2_ref[...],
                preferred_element_type=jnp.float32) + b2_ref[...]
    o_ref[...] = jnp.exp(s)


def _mlp(g2, m0, m1, W1, b1r, W2p, b2r, blk=256):
    NR, D2 = g2.shape
    D, H = W1.shape
    grid = NR // blk
    return pl.pallas_call(
        _mlp_body,
        grid=(grid,),
        in_specs=[
            pl.BlockSpec((blk, D2), lambda i: (i, 0)),
            pl.BlockSpec((blk, 1), lambda i: (i, 0)),
            pl.BlockSpec((blk, 1), lambda i: (i, 0)),
            pl.BlockSpec((D, H), lambda i: (0, 0)),
            pl.BlockSpec((1, H), lambda i: (0, 0)),
            pl.BlockSpec((H, _KP), lambda i: (0, 0)),
            pl.BlockSpec((1, _KP), lambda i: (0, 0)),
        ],
        out_specs=pl.BlockSpec((blk, _KP), lambda i: (i, 0)),
        out_shape=jax.ShapeDtypeStruct((NR, _KP), jnp.float32),
    )(g2, m0, m1, W1, b1r, W2p, b2r)


def _crf_body(K, T, s_ref, tr_ref, st_ref, en_ref, ln_ref, o_ref):
    ii = lax.broadcasted_iota(jnp.int32, (_KP, _KP), 0)
    jj = lax.broadcasted_iota(jnp.int32, (_KP, _KP), 1)
    E = jnp.where((ii < K) & (jj < K), jnp.exp(tr_ref[...]), 0.0)
    ln = ln_ref[...]
    w = jnp.exp(st_ref[...]) * s_ref[0]
    c = jnp.zeros((w.shape[0], 1), jnp.float32)
    ngrp = (T - 1 + _GRP - 1) // _GRP

    def group(gi, carry):
        w, c = carry
        t0 = 1 + gi * _GRP
        for k in range(_GRP):
            t = jnp.minimum(t0 + k, T - 1)
            wn = jnp.dot(w, E, preferred_element_type=jnp.float32) * s_ref[t]
            keep = (t0 + k).astype(jnp.float32) < ln
            w = jnp.where(keep, wn, w)
        s = jnp.max(w, axis=1, keepdims=True)
        return w / s, c + jnp.log(s)

    w, c = lax.fori_loop(0, ngrp, group, (w, c))
    z = jnp.sum(w * jnp.exp(en_ref[...]), axis=1, keepdims=True)
    o_ref[...] = c + jnp.log(z)


def _crf(s_t, trp, stp, enp, lnf, K):
    T, B, _ = s_t.shape
    return pl.pallas_call(
        functools.partial(_crf_body, K, T),
        out_shape=jax.ShapeDtypeStruct((B, 1), jnp.float32),
    )(s_t, trp, stp, enp, lnf)


def kernel(bert_outputs, token_idxs, token_masks, token_nums, W1, b1, W2, b2,
           transitions, start_trans, end_trans):
    B, L, D = bert_outputs.shape
    TW = token_idxs.shape[1]
    T = TW // 2
    K = W2.shape[1]
    kpad = _KP - K

    table = bert_outputs.reshape(B * L, D)
    idx2 = token_idxs.reshape(B * TW // _CH, _CH).astype(jnp.int32)
    gathered = _sc_gather(table, idx2, TW, L)

    g2 = gathered.reshape(B * T, 2 * D)
    m = token_masks.reshape(B * T, 2)
    W2p = jnp.pad(W2, ((0, 0), (0, kpad))).astype(jnp.bfloat16)
    b2r = jnp.pad(b2, (0, kpad)).reshape(1, _KP)
    escores = _mlp(g2, m[:, 0:1], m[:, 1:2], W1.astype(jnp.bfloat16),
                   b1.reshape(1, -1), W2p, b2r)

    s_t = escores.reshape(B, T, _KP).transpose(1, 0, 2)
    trp = jnp.pad(transitions, ((0, kpad), (0, kpad)))
    stp = jnp.pad(start_trans, (0, kpad), constant_values=_NEG).reshape(1, _KP)
    enp = jnp.pad(end_trans, (0, kpad), constant_values=_NEG).reshape(1, _KP)
    lnf = jnp.maximum(token_nums, 1).astype(jnp.float32).reshape(B, 1)
    logZ = _crf(s_t, trp, stp, enp, lnf, K)
    return logZ[:, 0]
